# baseline (speedup 1.0000x reference)
import functools

import jax
import jax.numpy as jnp
import numpy as np
from jax import lax
from jax.experimental import pallas as pl
from jax.experimental.pallas import tpu as pltpu
from jax.experimental.pallas import tpu_sc as plsc

NC = 2
NS = 16
NW = NC * NS

BATCH = 16384
D = 32
DP = D // 2
N_ROWS = 1000000
PW = 32768
P_GRID = (N_ROWS + PW - 1) // PW
SUB = PW // 4
QROWS = P_GRID * SUB
B_PER_W = BATCH // NW
CHUNK = 128
N_CHUNK = B_PER_W // CHUNK
IDX_ROWS = BATCH // CHUNK


def _sel_mats():
    k = np.arange(128)[:, None]
    c = np.arange(64)[None, :]
    e = ((k // 32 == c // 16) & (k % 32 == 2 * (c % 16))).astype(np.float32)
    o = ((k // 32 == c // 16) & (k % 32 == 2 * (c % 16) + 1)).astype(np.float32)
    return jnp.asarray(e, jnp.bfloat16), jnp.asarray(o, jnp.bfloat16)


def _pack_body(x0, x1, x2, x3, y0, y1, y2, y3, ee_ref, eo_ref,
               uo_ref, io_ref):
    ee, eo = ee_ref[...], eo_ref[...]
    xs = jnp.concatenate([x0[...], x1[...], x2[...], x3[...]],
                         axis=0).astype(jnp.bfloat16)
    ys = jnp.concatenate([y0[...], y1[...], y2[...], y3[...]],
                         axis=0).astype(jnp.bfloat16)
    dn = (((0,), (0,)), ((), ()))
    for src, dst in ((xs, uo_ref), (ys, io_ref)):
        a = lax.dot_general(src, ee, dn, preferred_element_type=jnp.float32)
        b = lax.dot_general(src, eo, dn, preferred_element_type=jnp.float32)
        ar = a.astype(jnp.bfloat16).astype(jnp.float32)
        br = b.astype(jnp.bfloat16).astype(jnp.float32)
        packed = pltpu.pack_elementwise([ar, br], packed_dtype=jnp.bfloat16)
        dst[...] = pltpu.bitcast(packed, jnp.float32)


def _tc_pack(ut_t, it_t):
    ee, eo = _sel_mats()
    line_t = jax.ShapeDtypeStruct((QROWS, 64), jnp.float32)
    last = N_ROWS // SUB
    sub = lambda g: pl.BlockSpec(
        (D, SUB), lambda i, g=g: (0, jnp.minimum(4 * i + g, last)))
    uq, iq = pl.pallas_call(
        _pack_body,
        grid=(P_GRID,),
        in_specs=[sub(0), sub(1), sub(2), sub(3),
                  sub(0), sub(1), sub(2), sub(3),
                  pl.BlockSpec((128, 64), lambda i: (0, 0)),
                  pl.BlockSpec((128, 64), lambda i: (0, 0))],
        out_specs=[pl.BlockSpec((SUB, 64), lambda i: (i, 0)),
                   pl.BlockSpec((SUB, 64), lambda i: (i, 0))],
        out_shape=[line_t, line_t],
    )(ut_t, ut_t, ut_t, ut_t, it_t, it_t, it_t, it_t, ee, eo)
    return uq.reshape(QROWS * 4, DP), iq.reshape(QROWS * 4, DP)


def _gather_body(ut_hbm, it_hbm, ui_hbm, ii_hbm, u_hbm, v_hbm,
                 uidx_v, iidx_v, urows_v, irows_v, sem):
    wid = lax.axis_index("s") * NC + lax.axis_index("c")
    row0 = wid * N_CHUNK
    pltpu.sync_copy(ui_hbm.at[pl.ds(row0, N_CHUNK)], uidx_v)
    pltpu.sync_copy(ii_hbm.at[pl.ds(row0, N_CHUNK)], iidx_v)
    copies = []
    for j in range(N_CHUNK):
        copies.append(pltpu.async_copy(
            ut_hbm.at[uidx_v.at[j]], urows_v.at[pl.ds(j * CHUNK, CHUNK)], sem))
        copies.append(pltpu.async_copy(
            it_hbm.at[iidx_v.at[j]], irows_v.at[pl.ds(j * CHUNK, CHUNK)], sem))
    for c in copies:
        c.wait()
    base = wid * B_PER_W
    pltpu.sync_copy(urows_v, u_hbm.at[pl.ds(base, B_PER_W)])
    pltpu.sync_copy(irows_v, v_hbm.at[pl.ds(base, B_PER_W)])


def _sc_gather(user_table, item_table, user_idx, item_idx):
    mesh = plsc.VectorSubcoreMesh(core_axis_name="c", subcore_axis_name="s")
    rows_t = jax.ShapeDtypeStruct((BATCH, DP), jnp.float32)
    k = pl.kernel(
        _gather_body,
        out_type=[rows_t, rows_t],
        mesh=mesh,
        compiler_params=pltpu.CompilerParams(use_tc_tiling_on_sc=False),
        scratch_types=[
            pltpu.VMEM((N_CHUNK, CHUNK), jnp.int32),
            pltpu.VMEM((N_CHUNK, CHUNK), jnp.int32),
            pltpu.VMEM((B_PER_W, DP), jnp.float32),
            pltpu.VMEM((B_PER_W, DP), jnp.float32),
            pltpu.SemaphoreType.DMA,
        ],
    )
    return k(user_table, item_table,
             user_idx.reshape(IDX_ROWS, CHUNK), item_idx.reshape(IDX_ROWS, CHUNK))


BLK = 2048


def _unpack(p_ref):
    pi = pltpu.bitcast(p_ref[...], jnp.int32)
    a = pltpu.unpack_elementwise(
        pi, index=0, packed_dtype=jnp.int16, unpacked_dtype=jnp.int32)
    b = pltpu.unpack_elementwise(
        pi, index=1, packed_dtype=jnp.int16, unpacked_dtype=jnp.int32)
    af = pltpu.bitcast(jnp.left_shift(a, 16), jnp.float32)
    bf = pltpu.bitcast(jnp.left_shift(b, 16), jnp.float32)
    return af, bf


def _mlp_body(u_ref, v_ref, w1ue_ref, w1uo_ref, w1ve_ref, w1vo_ref,
              b1_ref, w2_ref, b2_ref, wo_ref, bo_ref, o_ref):
    ua, ub = _unpack(u_ref)
    va, vb = _unpack(v_ref)
    h = (ua @ w1ue_ref[...] + ub @ w1uo_ref[...]
         + va @ w1ve_ref[...] + vb @ w1vo_ref[...] + b1_ref[...])
    h = jnp.maximum(h, 0.0)
    h = jnp.maximum(h @ w2_ref[...] + b2_ref[...], 0.0)
    o_ref[...] = h @ wo_ref[...] + bo_ref[...]


def _tc_mlp(u, v, W1, b1, W2, b2, Wo, bo):
    w1u, w1v = W1[:D], W1[D:]
    w1ue, w1uo = w1u[0::2], w1u[1::2]
    w1ve, w1vo = w1v[0::2], w1v[1::2]
    grid = (BATCH // BLK,)
    full = lambda shape: pl.BlockSpec(shape, lambda i: (0, 0))
    out = pl.pallas_call(
        _mlp_body,
        grid=grid,
        in_specs=[
            pl.BlockSpec((BLK, DP), lambda i: (i, 0)),
            pl.BlockSpec((BLK, DP), lambda i: (i, 0)),
            full((DP, 64)),
            full((DP, 64)),
            full((DP, 64)),
            full((DP, 64)),
            full((1, 64)),
            full((64, 32)),
            full((1, 32)),
            full((32, 1)),
            full((1, 1)),
        ],
        out_specs=pl.BlockSpec((BLK, 1), lambda i: (i, 0)),
        out_shape=jax.ShapeDtypeStruct((BATCH, 1), jnp.float32),
    )(u, v, w1ue, w1uo, w1ve, w1vo, b1.reshape(1, 64), W2,
      b2.reshape(1, 32), Wo, bo.reshape(1, 1))
    return out[:, 0]


def kernel(user_indices, item_indices, user_table, item_table,
           W1, b1, W2, b2, Wo, bo):
    uq, iq = _tc_pack(user_table.T, item_table.T)

    def remap(r):
        i = r // PW
        rem = r % PW
        return 4 * (SUB * i + rem % SUB) + rem // SUB

    uidx = remap(user_indices)
    iidx = remap(item_indices)
    u, v = _sc_gather(uq, iq, uidx, iidx)
    return _tc_mlp(u, v, W1, b1, W2, b2, Wo, bo)
